# baseline (speedup 1.0000x reference)
import functools

import jax
import jax.numpy as jnp
from jax import lax
from jax.experimental import pallas as pl
from jax.experimental.pallas import tpu as pltpu
from jax.experimental.pallas import tpu_sc as plsc

D = 128
L = 16
NSUB = 16
NSC = 2
G = 256
GC = 512
EBLK = 512
SENTINEL = 1 << 30

N_U, N_B, N_C = 50000, 50000, 10000
CH_UB = 6272
NCH_UB = 8
CH_C = 5120
NCH_C = 2
NP_UB = CH_UB * NCH_UB
NP_C = CH_C * NCH_C
EP_UB = 163840
EP_UU = 114688
EP_CB = 65536
BLK = 512

_MESH = dict(core_axis_name="c", subcore_axis_name="s",
             num_cores=NSC, num_subcores=NSUB)
_CPARAMS = pltpu.CompilerParams(needs_layout_passes=False)


@functools.lru_cache(maxsize=None)
def _make_seg_kernel(e_pad: int, ch: int, nch: int):
    n_sub = e_pad // NSUB
    nsb = n_sub // EBLK
    assert nsb % 2 == 0
    rows_share = ch // NSUB
    n_pad = ch * nch
    cap = G + EBLK + L
    tailc = (cap - G + L - 1) // L

    out_type = jax.ShapeDtypeStruct((n_pad, D), jnp.float32)
    scratch = [
        pltpu.VMEM_SHARED((ch + L, D), jnp.float32),
        pltpu.VMEM((2, EBLK), jnp.int32),
        pltpu.VMEM((2, EBLK), jnp.int32),
        pltpu.VMEM((cap,), jnp.int32),
        pltpu.VMEM((cap,), jnp.int32),
        pltpu.VMEM((G,), jnp.int32),
        pltpu.VMEM((G,), jnp.int32),
        pltpu.VMEM((G,), jnp.int32),
        pltpu.VMEM((G,), jnp.int32),
        pltpu.VMEM((G, D), jnp.float32),
        pltpu.VMEM((G, D), jnp.float32),
        pltpu.VMEM((32, D), jnp.float32),
        pltpu.SemaphoreType.DMA,
        pltpu.SemaphoreType.DMA,
        pltpu.SemaphoreType.DMA,
        pltpu.SemaphoreType.DMA,
    ]

    def body(x_hbm, e_hbm, s_hbm,
             acc, ebuf0, ebuf1, sel_src, sel_dst,
             gidx0, gidx1, sidx0, sidx1, rows0, rows1, zbuf,
             esem0, esem1, sem0, sem1):
        cid = lax.axis_index("c")
        sid = lax.axis_index("s")
        base = sid * n_sub

        z16 = jnp.zeros((L,), jnp.float32)
        zi16 = jnp.zeros((L,), jnp.int32)
        pad_dst = jnp.full((L,), ch, jnp.int32)

        @pl.loop(0, 32)
        def _(i):
            for j in range(D // L):
                zbuf[i, pl.ds(j * L, L)] = z16

        def edma(t, buf, sem):
            pltpu.async_copy(e_hbm.at[:, pl.ds(base + t * EBLK, EBLK)],
                             buf, sem)

        def ewait(t, buf, sem):
            pltpu.make_async_copy(e_hbm.at[:, pl.ds(base + t * EBLK, EBLK)],
                                  buf, sem).wait()

        def issue(gidx, sidx, rows, sem):
            for j in range(G // L):
                gidx[pl.ds(j * L, L)] = sel_src[pl.ds(j * L, L)]
                sidx[pl.ds(j * L, L)] = sel_dst[pl.ds(j * L, L)]
            pltpu.async_copy(x_hbm.at[gidx], rows, sem)

        def finish(gidx, sidx, rows, sem):
            pltpu.make_async_copy(x_hbm.at[gidx], rows, sem).wait()
            pltpu.sync_copy(rows, acc.at[sidx], add=True)

        def compress_fire(ebuf, lo, carry):
            def blk(i, w):
                s16 = ebuf[0, pl.ds(i * L, L)]
                d16 = ebuf[1, pl.ds(i * L, L)]
                dl = d16 - lo
                m = (dl >= 0) & (dl < ch)
                plsc.store_compressed(sel_src.at[pl.ds(w, L)], s16, mask=m)
                plsc.store_compressed(sel_dst.at[pl.ds(w, L)], dl, mask=m)
                return w + jnp.sum(m.astype(jnp.int32))

            w = lax.fori_loop(0, EBLK // L, blk, carry[0])

            def fcond(c):
                return c[0] >= G

            def fbody(c):
                w, pend = c
                cur = jnp.where(pend == 1, 2, 1)

                @pl.when(cur == 1)
                def _():
                    issue(gidx0, sidx0, rows0, sem0)

                @pl.when(cur == 2)
                def _():
                    issue(gidx1, sidx1, rows1, sem1)

                for j in range(tailc):
                    sel_src[pl.ds(j * L, L)] = sel_src[pl.ds(G + j * L, L)]
                    sel_dst[pl.ds(j * L, L)] = sel_dst[pl.ds(G + j * L, L)]

                @pl.when(pend == 1)
                def _():
                    finish(gidx0, sidx0, rows0, sem0)

                @pl.when(pend == 2)
                def _():
                    finish(gidx1, sidx1, rows1, sem1)

                return (w - G, cur)

            return lax.while_loop(fcond, fbody, (w, carry[1]))

        for k in range(nch // NSC):
            chunk = NSC * k + cid
            lo = chunk * ch
            row0 = sid * rows_share

            rem, t = rows_share, 0
            while rem > 0:
                n = min(32, rem)
                pltpu.sync_copy(zbuf.at[pl.ds(0, n)],
                                acc.at[pl.ds(row0 + t * 32, n)])
                rem -= n
                t += 1

            @pl.when(sid == NSUB - 1)
            def _():
                pltpu.sync_copy(zbuf.at[pl.ds(0, L)], acc.at[pl.ds(ch, L)])

            plsc.subcore_barrier()

            edma(0, ebuf0, esem0)

            def sblk(tt, carry):
                t0 = 2 * tt
                ewait(t0, ebuf0, esem0)
                edma(t0 + 1, ebuf1, esem1)
                carry = compress_fire(ebuf0, lo, carry)
                ewait(t0 + 1, ebuf1, esem1)

                @pl.when(t0 + 2 < nsb)
                def _():
                    edma(t0 + 2, ebuf0, esem0)

                return compress_fire(ebuf1, lo, carry)

            w, pend = lax.fori_loop(0, nsb // 2, sblk,
                                    (jnp.int32(0), jnp.int32(0)))

            @pl.when(pend == 1)
            def _():
                finish(gidx0, sidx0, rows0, sem0)

            @pl.when(pend == 2)
            def _():
                finish(gidx1, sidx1, rows1, sem1)

            @pl.when(w > 0)
            def _():
                for j in range(G // L):
                    sel_src[pl.ds(w + j * L, L)] = zi16
                    sel_dst[pl.ds(w + j * L, L)] = pad_dst
                issue(gidx0, sidx0, rows0, sem0)
                finish(gidx0, sidx0, rows0, sem0)

            plsc.subcore_barrier()

            pltpu.sync_copy(acc.at[pl.ds(row0, rows_share)],
                            s_hbm.at[pl.ds(lo + row0, rows_share)])
            plsc.subcore_barrier()

    return pl.kernel(body, out_type=out_type,
                     mesh=plsc.VectorSubcoreMesh(**_MESH),
                     scratch_types=scratch, compiler_params=_CPARAMS)


@functools.lru_cache(maxsize=None)
def _make_cnt_kernel(e_pad: int, ch: int, nch: int):
    n_sub = e_pad // NSUB
    nsb = n_sub // EBLK
    assert nsb % 2 == 0
    rows_share = ch // NSUB
    n_pad = ch * nch
    cap = GC + EBLK + L
    tailc = (cap - GC + L - 1) // L

    out_type = jax.ShapeDtypeStruct((n_pad, D), jnp.float32)
    scratch = [
        pltpu.VMEM_SHARED((ch + L, D), jnp.float32),
        pltpu.VMEM((2, EBLK), jnp.int32),
        pltpu.VMEM((2, EBLK), jnp.int32),
        pltpu.VMEM((cap,), jnp.int32),
        pltpu.VMEM((GC,), jnp.int32),
        pltpu.VMEM((GC, D), jnp.float32),
        pltpu.VMEM((32, D), jnp.float32),
        pltpu.SemaphoreType.DMA,
        pltpu.SemaphoreType.DMA,
    ]

    def body(e_hbm, cnt_hbm,
             cacc, ebuf0, ebuf1, sel_dst, sidx, ones, zcnt, esem0, esem1):
        cid = lax.axis_index("c")
        sid = lax.axis_index("s")
        base = sid * n_sub

        z16 = jnp.zeros((L,), jnp.float32)
        one16 = jnp.ones((L,), jnp.float32)
        pad_dst = jnp.full((L,), ch, jnp.int32)

        @pl.loop(0, GC)
        def _(i):
            for j in range(D // L):
                ones[i, pl.ds(j * L, L)] = one16

        @pl.loop(0, 32)
        def _(i):
            for j in range(D // L):
                zcnt[i, pl.ds(j * L, L)] = z16

        def edma(t, buf, sem):
            pltpu.async_copy(e_hbm.at[:, pl.ds(base + t * EBLK, EBLK)],
                             buf, sem)

        def ewait(t, buf, sem):
            pltpu.make_async_copy(e_hbm.at[:, pl.ds(base + t * EBLK, EBLK)],
                                  buf, sem).wait()

        def fire():
            for j in range(GC // L):
                sidx[pl.ds(j * L, L)] = sel_dst[pl.ds(j * L, L)]
            pltpu.sync_copy(ones, cacc.at[sidx], add=True)

        def compress_fire(ebuf, lo, w):
            def blk(i, w):
                d16 = ebuf[1, pl.ds(i * L, L)]
                dl = d16 - lo
                m = (dl >= 0) & (dl < ch)
                plsc.store_compressed(sel_dst.at[pl.ds(w, L)], dl, mask=m)
                return w + jnp.sum(m.astype(jnp.int32))

            w = lax.fori_loop(0, EBLK // L, blk, w)

            def fbody(w):
                fire()
                for j in range(tailc):
                    sel_dst[pl.ds(j * L, L)] = sel_dst[pl.ds(GC + j * L, L)]
                return w - GC

            return lax.while_loop(lambda w: w >= GC, fbody, w)

        for k in range(nch // NSC):
            chunk = NSC * k + cid
            lo = chunk * ch
            row0 = sid * rows_share

            rem, t = rows_share, 0
            while rem > 0:
                n = min(32, rem)
                pltpu.sync_copy(zcnt.at[pl.ds(0, n)],
                                cacc.at[pl.ds(row0 + t * 32, n)])
                rem -= n
                t += 1

            @pl.when(sid == NSUB - 1)
            def _():
                pltpu.sync_copy(zcnt.at[pl.ds(0, L)], cacc.at[pl.ds(ch, L)])

            plsc.subcore_barrier()

            edma(0, ebuf0, esem0)

            def sblk(tt, w):
                t0 = 2 * tt
                ewait(t0, ebuf0, esem0)
                edma(t0 + 1, ebuf1, esem1)
                w = compress_fire(ebuf0, lo, w)
                ewait(t0 + 1, ebuf1, esem1)

                @pl.when(t0 + 2 < nsb)
                def _():
                    edma(t0 + 2, ebuf0, esem0)

                return compress_fire(ebuf1, lo, w)

            w = lax.fori_loop(0, nsb // 2, sblk, jnp.int32(0))

            @pl.when(w > 0)
            def _():
                for j in range(GC // L):
                    sel_dst[pl.ds(w + j * L, L)] = pad_dst
                fire()

            plsc.subcore_barrier()

            pltpu.sync_copy(cacc.at[pl.ds(row0, rows_share)],
                            cnt_hbm.at[pl.ds(lo + row0, rows_share)])
            plsc.subcore_barrier()

    return pl.kernel(body, out_type=out_type,
                     mesh=plsc.VectorSubcoreMesh(**_MESH),
                     scratch_types=scratch, compiler_params=_CPARAMS)


@functools.lru_cache(maxsize=None)
def _make_dense_kernel(n_pad: int, n_rel: int):
    nb = n_pad // BLK
    R = n_rel

    def body(x_ref, *refs):
        s_refs = refs[0:R]
        cnt_refs = refs[R:2 * R]
        wl_refs = refs[2 * R:3 * R]
        wr_refs = refs[3 * R:4 * R]
        bl_refs = refs[4 * R:5 * R]
        out_ref = refs[5 * R]

        wr = wr_refs[0][...]
        for r in range(1, R):
            wr = wr + wr_refs[r][...]
        b = bl_refs[0][...]
        for r in range(1, R):
            b = b + bl_refs[r][...]

        acc = jnp.dot(x_ref[...], wr, preferred_element_type=jnp.float32,
                      precision=lax.Precision.HIGHEST)
        for r in range(R):
            c = cnt_refs[r][:, 0:1]
            mean = s_refs[r][...] * (1.0 / jnp.maximum(c, 1.0))
            acc = acc + jnp.dot(mean, wl_refs[r][...],
                                preferred_element_type=jnp.float32,
                                precision=lax.Precision.HIGHEST)
        out_ref[...] = jnp.maximum(acc + b, 0.0)

    row_spec = pl.BlockSpec((BLK, D), lambda i: (i, 0))
    w_spec = pl.BlockSpec((D, D), lambda i: (0, 0))
    b_spec = pl.BlockSpec((1, D), lambda i: (0, 0))

    return pl.pallas_call(
        body,
        grid=(nb,),
        in_specs=[row_spec] + [row_spec] * R + [row_spec] * R
                 + [w_spec] * R + [w_spec] * R + [b_spec] * R,
        out_specs=row_spec,
        out_shape=jax.ShapeDtypeStruct((n_pad, D), jnp.float32),
    )


_REL_CFG = {
    "ub": (EP_UB, CH_UB, NCH_UB),
    "bu": (EP_UB, CH_UB, NCH_UB),
    "uu": (EP_UU, CH_UB, NCH_UB),
    "bb": (EP_UU, CH_UB, NCH_UB),
    "cb": (EP_CB, CH_UB, NCH_UB),
    "bc": (EP_CB, CH_C, NCH_C),
}


def _prep_edges(ei, e_pad):
    e = ei.shape[1]
    return jnp.pad(ei, ((0, 0), (0, e_pad - e)),
                   constant_values=SENTINEL).astype(jnp.int32)


def _layer(xu, xb, xc, edges, cnt, params, pre):
    xs = {"ub": xu, "bu": xb, "uu": xu, "bb": xb, "cb": xc, "bc": xb}

    s = {}
    for r in ("bu", "uu", "ub", "bb", "cb", "bc"):
        e_pad, ch, nch = _REL_CFG[r]
        s[r] = _make_seg_kernel(e_pad, ch, nch)(xs[r], edges[r])

    def dense(n_pad, x_dst, rels):
        kern = _make_dense_kernel(n_pad, len(rels))
        args = [x_dst]
        args += [s[r] for r in rels]
        args += [cnt[r] for r in rels]
        args += [params[pre + "_" + r + "_Wl"] for r in rels]
        args += [params[pre + "_" + r + "_Wr"] for r in rels]
        args += [params[pre + "_" + r + "_bl"].reshape(1, D) for r in rels]
        return kern(*args)

    ou = dense(NP_UB, xu, ("bu", "uu"))
    ob = dense(NP_UB, xb, ("ub", "bb", "cb"))
    oc = dense(NP_C, xc, ("bc",))
    return ou, ob, oc


def kernel(x_user, x_business, x_category, ei_interacts, ei_rev_interacts,
           ei_friends, ei_similar, ei_belongs, ei_category_of, params):
    xu = jnp.pad(x_user, ((0, NP_UB - N_U), (0, 0)))
    xb = jnp.pad(x_business, ((0, NP_UB - N_B), (0, 0)))
    xc = jnp.pad(x_category, ((0, NP_C - N_C), (0, 0)))

    eis = {"ub": ei_interacts, "bu": ei_rev_interacts, "uu": ei_friends,
           "bb": ei_similar, "bc": ei_belongs, "cb": ei_category_of}
    edges = {r: _prep_edges(eis[r], _REL_CFG[r][0]) for r in eis}

    cnt = {}
    for r in ("bu", "uu", "ub", "bb", "cb", "bc"):
        e_pad, ch, nch = _REL_CFG[r]
        cnt[r] = _make_cnt_kernel(e_pad, ch, nch)(edges[r])

    xu, xb, xc = _layer(xu, xb, xc, edges, cnt, params, "l1")
    xu, xb, xc = _layer(xu, xb, xc, edges, cnt, params, "l2")
    return (xu[:N_U], xb[:N_B], xc[:N_C])


# speedup vs baseline: 2.5731x; 2.5731x over previous
import functools

import jax
import jax.numpy as jnp
from jax import lax
from jax.experimental import pallas as pl
from jax.experimental.pallas import tpu as pltpu
from jax.experimental.pallas import tpu_sc as plsc

D = 128
L = 16
NSUB = 16
NSC = 2
G = 128
GS = 64
EBLK = 512
SENTINEL = 1 << 30

N_U, N_B, N_C = 50000, 50000, 10000
CH_UB = 12544
CH_C = 5120
NP_UB = CH_UB * 4
NP_C = CH_C * 2
EP_UB = 155648
EP_UU = 106496
EP_CB = 57344
BLK = 512

_MESH = dict(core_axis_name="c", subcore_axis_name="s",
             num_cores=NSC, num_subcores=NSUB)


@functools.lru_cache(maxsize=None)
def _make_seg_kernel(e_pad: int, ch: int, nch: int):
    n_sub = e_pad // NSUB
    nsb = n_sub // EBLK
    rows_share = ch // NSUB
    n_pad = ch * nch
    out_type = jax.ShapeDtypeStruct((n_pad, D), jnp.float32)
    scratch = [
        pltpu.VMEM_SHARED((ch + L, D), jnp.float32),
        pltpu.VMEM((EBLK,), jnp.int32),
        pltpu.VMEM((EBLK,), jnp.int32),
        pltpu.VMEM((2 * GS,), jnp.int32),
        pltpu.VMEM((2 * GS,), jnp.int32),
        pltpu.VMEM((GS,), jnp.int32),
        pltpu.VMEM((GS,), jnp.int32),
        pltpu.VMEM((GS,), jnp.int32),
        pltpu.VMEM((GS,), jnp.int32),
        pltpu.VMEM((GS, D), jnp.float32),
        pltpu.VMEM((GS, D), jnp.float32),
        pltpu.VMEM((32, D), jnp.float32),
        pltpu.SemaphoreType.DMA,
        pltpu.SemaphoreType.DMA,
    ]

    def body(x_hbm, src_hbm, dst_hbm, s_hbm,
             acc, src_blk, dst_blk, sel_src, sel_dst,
             gidx0, gidx1, sidx0, sidx1, rows0, rows1, zbuf, sem0, sem1):
        cid = lax.axis_index("c")
        sid = lax.axis_index("s")
        base = sid * n_sub
        z16 = jnp.zeros((L,), jnp.float32)
        zi16 = jnp.zeros((L,), jnp.int32)
        pad_dst = jnp.full((L,), ch, jnp.int32)

        @pl.loop(0, 32)
        def _(i):
            for j in range(D // L):
                zbuf[i, pl.ds(j * L, L)] = z16

        def issue(gidx, sidx, rows, sem):
            for j in range(GS // L):
                gidx[pl.ds(j * L, L)] = sel_src[pl.ds(j * L, L)]
                sidx[pl.ds(j * L, L)] = sel_dst[pl.ds(j * L, L)]
            pltpu.async_copy(x_hbm.at[gidx], rows, sem)

        def finish(gidx, sidx, rows, sem):
            pltpu.make_async_copy(x_hbm.at[gidx], rows, sem).wait()
            pltpu.sync_copy(rows, acc.at[sidx], add=True)

        for k in range(nch // NSC):
            chunk = NSC * k + cid
            lo = chunk * ch
            row0 = sid * rows_share
            rem, t = rows_share, 0
            while rem > 0:
                n = min(32, rem)
                pltpu.sync_copy(zbuf.at[pl.ds(0, n)],
                                acc.at[pl.ds(row0 + t * 32, n)])
                rem -= n
                t += 1

            @pl.when(sid == NSUB - 1)
            def _():
                pltpu.sync_copy(zbuf.at[pl.ds(0, L)], acc.at[pl.ds(ch, L)])

            plsc.subcore_barrier()

            def sblk(t, carry):
                pltpu.sync_copy(src_hbm.at[pl.ds(base + t * EBLK, EBLK)],
                                src_blk)
                pltpu.sync_copy(dst_hbm.at[pl.ds(base + t * EBLK, EBLK)],
                                dst_blk)

                def blk(i, carry):
                    w, pend = carry
                    s16 = src_blk[pl.ds(i * L, L)]
                    d16 = dst_blk[pl.ds(i * L, L)]
                    dl = d16 - lo
                    m = (dl >= 0) & (dl < ch)
                    plsc.store_compressed(sel_src.at[pl.ds(w, L)], s16,
                                          mask=m)
                    plsc.store_compressed(sel_dst.at[pl.ds(w, L)], dl,
                                          mask=m)
                    w = w + jnp.sum(m.astype(jnp.int32))
                    full = w >= GS
                    cur = jnp.where(full, jnp.where(pend == 1, 2, 1), pend)

                    @pl.when(full & (cur == 1))
                    def _():
                        issue(gidx0, sidx0, rows0, sem0)

                    @pl.when(full & (cur == 2))
                    def _():
                        issue(gidx1, sidx1, rows1, sem1)

                    @pl.when(full)
                    def _():
                        sel_src[pl.ds(0, L)] = sel_src[pl.ds(GS, L)]
                        sel_dst[pl.ds(0, L)] = sel_dst[pl.ds(GS, L)]

                    @pl.when(full & (pend == 1))
                    def _():
                        finish(gidx0, sidx0, rows0, sem0)

                    @pl.when(full & (pend == 2))
                    def _():
                        finish(gidx1, sidx1, rows1, sem1)

                    return (jnp.where(full, w - GS, w), cur)

                return lax.fori_loop(0, EBLK // L, blk, carry)

            w, pend = lax.fori_loop(0, nsb, sblk,
                                    (jnp.int32(0), jnp.int32(0)))

            @pl.when(pend == 1)
            def _():
                finish(gidx0, sidx0, rows0, sem0)

            @pl.when(pend == 2)
            def _():
                finish(gidx1, sidx1, rows1, sem1)

            @pl.when(w > 0)
            def _():
                for j in range(GS // L):
                    sel_src[pl.ds(w + j * L, L)] = zi16
                    sel_dst[pl.ds(w + j * L, L)] = pad_dst
                issue(gidx0, sidx0, rows0, sem0)
                finish(gidx0, sidx0, rows0, sem0)

            plsc.subcore_barrier()
            pltpu.sync_copy(acc.at[pl.ds(row0, rows_share)],
                            s_hbm.at[pl.ds(lo + row0, rows_share)])
            plsc.subcore_barrier()

    return pl.kernel(body, out_type=out_type,
                     mesh=plsc.VectorSubcoreMesh(**_MESH),
                     scratch_types=scratch, compiler_params=pltpu.CompilerParams(
                         needs_layout_passes=False))


@functools.lru_cache(maxsize=None)
def _make_cnt_kernel(e_pad: int, ch: int, nch: int):
    n_sub = e_pad // NSUB
    nsb = n_sub // EBLK
    rows_share = ch // NSUB
    n_pad = ch * nch

    out_type = jax.ShapeDtypeStruct((n_pad, D), jnp.float32)
    scratch = [
        pltpu.VMEM_SHARED((ch + L, D), jnp.float32),
        pltpu.VMEM((EBLK,), jnp.int32),
        pltpu.VMEM((2 * G,), jnp.int32),
        pltpu.VMEM((G,), jnp.int32),
        pltpu.VMEM((G, D), jnp.float32),
        pltpu.VMEM((32, D), jnp.float32),
    ]

    def body(dst_hbm, cnt_hbm, cacc, dst_blk, sel_dst, sidx, ones, zcnt):
        cid = lax.axis_index("c")
        sid = lax.axis_index("s")
        base = sid * n_sub

        z16 = jnp.zeros((L,), jnp.float32)
        one16 = jnp.ones((L,), jnp.float32)
        pad_dst = jnp.full((L,), ch, jnp.int32)

        @pl.loop(0, G)
        def _(i):
            for j in range(D // L):
                ones[i, pl.ds(j * L, L)] = one16

        @pl.loop(0, 32)
        def _(i):
            for j in range(D // L):
                zcnt[i, pl.ds(j * L, L)] = z16

        def fire():
            for j in range(G // L):
                sidx[pl.ds(j * L, L)] = sel_dst[pl.ds(j * L, L)]
            pltpu.sync_copy(ones, cacc.at[sidx], add=True)

        for k in range(nch // NSC):
            chunk = NSC * k + cid
            lo = chunk * ch
            row0 = sid * rows_share

            rem, t = rows_share, 0
            while rem > 0:
                n = min(32, rem)
                pltpu.sync_copy(zcnt.at[pl.ds(0, n)],
                                cacc.at[pl.ds(row0 + t * 32, n)])
                rem -= n
                t += 1

            @pl.when(sid == NSUB - 1)
            def _():
                pltpu.sync_copy(zcnt.at[pl.ds(0, L)], cacc.at[pl.ds(ch, L)])

            plsc.subcore_barrier()

            def sblk(t, w):
                pltpu.sync_copy(dst_hbm.at[pl.ds(base + t * EBLK, EBLK)],
                                dst_blk)

                def blk(i, w):
                    d16 = dst_blk[pl.ds(i * L, L)]
                    dl = d16 - lo
                    m = (dl >= 0) & (dl < ch)
                    plsc.store_compressed(sel_dst.at[pl.ds(w, L)], dl,
                                          mask=m)
                    w = w + jnp.sum(m.astype(jnp.int32))

                    @pl.when(w >= G)
                    def _():
                        fire()
                        sel_dst[pl.ds(0, L)] = sel_dst[pl.ds(G, L)]

                    return jnp.where(w >= G, w - G, w)

                return lax.fori_loop(0, EBLK // L, blk, w)

            w = lax.fori_loop(0, nsb, sblk, jnp.int32(0))

            @pl.when(w > 0)
            def _():
                for j in range(G // L):
                    sel_dst[pl.ds(w + j * L, L)] = pad_dst
                fire()

            plsc.subcore_barrier()

            pltpu.sync_copy(cacc.at[pl.ds(row0, rows_share)],
                            cnt_hbm.at[pl.ds(lo + row0, rows_share)])
            plsc.subcore_barrier()

    return pl.kernel(body, out_type=out_type,
                     mesh=plsc.VectorSubcoreMesh(**_MESH),
                     scratch_types=scratch,
                     compiler_params=pltpu.CompilerParams(
                         needs_layout_passes=False))


@functools.lru_cache(maxsize=None)
def _make_dense_kernel(n_pad: int, n_rel: int):
    nb = n_pad // BLK
    R = n_rel

    def body(x_ref, *refs):
        s_refs = refs[0:R]
        cnt_refs = refs[R:2 * R]
        wl_refs = refs[2 * R:3 * R]
        wr_refs = refs[3 * R:4 * R]
        bl_refs = refs[4 * R:5 * R]
        out_ref = refs[5 * R]

        wr = wr_refs[0][...]
        for r in range(1, R):
            wr = wr + wr_refs[r][...]
        b = bl_refs[0][...]
        for r in range(1, R):
            b = b + bl_refs[r][...]

        acc = jnp.dot(x_ref[...], wr, preferred_element_type=jnp.float32,
                      precision=lax.Precision.HIGHEST)
        for r in range(R):
            c = cnt_refs[r][:, 0:1]
            mean = s_refs[r][...] * (1.0 / jnp.maximum(c, 1.0))
            acc = acc + jnp.dot(mean, wl_refs[r][...],
                                preferred_element_type=jnp.float32,
                                precision=lax.Precision.HIGHEST)
        out_ref[...] = jnp.maximum(acc + b, 0.0)

    row_spec = pl.BlockSpec((BLK, D), lambda i: (i, 0))
    cnt_spec = pl.BlockSpec((BLK, D), lambda i: (i, 0))
    w_spec = pl.BlockSpec((D, D), lambda i: (0, 0))
    b_spec = pl.BlockSpec((1, D), lambda i: (0, 0))

    return pl.pallas_call(
        body,
        grid=(nb,),
        in_specs=[row_spec] + [row_spec] * R + [cnt_spec] * R
                 + [w_spec] * R + [w_spec] * R + [b_spec] * R,
        out_specs=row_spec,
        out_shape=jax.ShapeDtypeStruct((n_pad, D), jnp.float32),
    )


_REL_CFG = {
    "ub": (EP_UB, CH_UB, 4),
    "bu": (EP_UB, CH_UB, 4),
    "uu": (EP_UU, CH_UB, 4),
    "bb": (EP_UU, CH_UB, 4),
    "cb": (EP_CB, CH_UB, 4),
    "bc": (EP_CB, CH_C, 2),
}


def _prep_edges(ei, e_pad):
    e = ei.shape[1]
    src = jnp.pad(ei[0], (0, e_pad - e))
    dst = jnp.pad(ei[1], (0, e_pad - e), constant_values=SENTINEL)
    return src, dst


def _layer(xu, xb, xc, edges, cnt, params, pre):
    xs = {"ub": xu, "bu": xb, "uu": xu, "bb": xb, "cb": xc, "bc": xb}

    s = {}
    for r in ("bu", "uu", "ub", "bb", "cb", "bc"):
        e_pad, ch, nch = _REL_CFG[r]
        s[r] = _make_seg_kernel(e_pad, ch, nch)(xs[r], *edges[r])

    def dense(n_pad, x_dst, rels):
        kern = _make_dense_kernel(n_pad, len(rels))
        args = [x_dst]
        args += [s[r] for r in rels]
        args += [cnt[r] for r in rels]
        args += [params[pre + "_" + r + "_Wl"] for r in rels]
        args += [params[pre + "_" + r + "_Wr"] for r in rels]
        args += [params[pre + "_" + r + "_bl"].reshape(1, D) for r in rels]
        return kern(*args)

    ou = dense(NP_UB, xu, ("bu", "uu"))
    ob = dense(NP_UB, xb, ("ub", "bb", "cb"))
    oc = dense(NP_C, xc, ("bc",))
    return ou, ob, oc


def kernel(x_user, x_business, x_category, ei_interacts, ei_rev_interacts,
           ei_friends, ei_similar, ei_belongs, ei_category_of, params):
    xu = jnp.pad(x_user, ((0, NP_UB - N_U), (0, 0)))
    xb = jnp.pad(x_business, ((0, NP_UB - N_B), (0, 0)))
    xc = jnp.pad(x_category, ((0, NP_C - N_C), (0, 0)))

    eis = {"ub": ei_interacts, "bu": ei_rev_interacts, "uu": ei_friends,
           "bb": ei_similar, "bc": ei_belongs, "cb": ei_category_of}
    edges = {r: _prep_edges(eis[r], _REL_CFG[r][0]) for r in eis}

    cnt = {}
    for r in ("bu", "uu", "ub", "bb", "cb", "bc"):
        e_pad, ch, nch = _REL_CFG[r]
        cnt[r] = _make_cnt_kernel(e_pad, ch, nch)(edges[r][1])

    xu, xb, xc = _layer(xu, xb, xc, edges, cnt, params, "l1")
    xu, xb, xc = _layer(xu, xb, xc, edges, cnt, params, "l2")
    return (xu[:N_U], xb[:N_B], xc[:N_C])
